# baseline (speedup 1.0000x reference)
import jax
import jax.numpy as jnp
from jax.experimental import pallas as pl
from jax.experimental.pallas import tpu as pltpu

_ORDER = tuple(range(17))

_B, _N, _J, _D = 32, 243, 17, 256
_TOTAL = _B * _N * _J * _D
_NCHUNK = 64
_CHUNK = _TOTAL // _NCHUNK
_NBUF = 8
_LOOKAHEAD = 4


def _ring_copy(x_hbm, o_hbm, bufs, in_sems, out_sems):
    def in_cp(g):
        k = g % _NBUF
        return pltpu.make_async_copy(
            x_hbm.at[pl.ds(g * _CHUNK, _CHUNK)], bufs.at[k], in_sems.at[k])

    def out_cp(g):
        k = g % _NBUF
        return pltpu.make_async_copy(
            bufs.at[k], o_hbm.at[pl.ds(g * _CHUNK, _CHUNK)], out_sems.at[k])

    for j in range(_LOOKAHEAD):
        in_cp(j).start()
    for g in range(_NCHUNK):
        a = g + _LOOKAHEAD
        if a < _NCHUNK:
            if a >= _NBUF:
                out_cp(a - _NBUF).wait()
            in_cp(a).start()
        in_cp(g).wait()
        out_cp(g).start()
    for g in range(_NCHUNK - _NBUF, _NCHUNK):
        out_cp(g).wait()


def kernel(x):
    flat = x.reshape(_TOTAL)
    out = pl.pallas_call(
        _ring_copy,
        in_specs=[pl.BlockSpec(memory_space=pl.ANY)],
        out_specs=pl.BlockSpec(memory_space=pl.ANY),
        out_shape=jax.ShapeDtypeStruct((_TOTAL,), jnp.float32),
        scratch_shapes=[
            pltpu.VMEM((_NBUF, _CHUNK), jnp.float32),
            pltpu.SemaphoreType.DMA((_NBUF,)),
            pltpu.SemaphoreType.DMA((_NBUF,)),
        ],
    )(flat)
    return out.reshape(_B, _N, _J, _D)


# speedup vs baseline: 5.0692x; 5.0692x over previous
import jax
import jax.numpy as jnp
from jax.experimental import pallas as pl

_ORDER = (0, 1, 2, 3, 4, 5, 6, 7, 8, 9, 10, 11, 12, 13, 14, 15, 16)
_IS_IDENTITY = _ORDER == tuple(range(len(_ORDER)))


def _reorder_block(x_ref, o_ref):
    if _IS_IDENTITY:
        o_ref[...] = x_ref[...]
    else:
        for j, s in enumerate(_ORDER):
            o_ref[:, :, j, :] = x_ref[:, :, s, :]


def kernel(x):
    b, n, j, d = x.shape
    grid = (b // 2,)
    return pl.pallas_call(
        _reorder_block,
        grid=grid,
        in_specs=[pl.BlockSpec((2, n, j, d), lambda i: (i, 0, 0, 0))],
        out_specs=pl.BlockSpec((2, n, j, d), lambda i: (i, 0, 0, 0)),
        out_shape=jax.ShapeDtypeStruct((b, n, j, d), x.dtype),
    )(x)


# speedup vs baseline: 19.4807x; 3.8429x over previous
import jax
import jax.numpy as jnp
from jax.experimental import pallas as pl

_ORDER = (0, 1, 2, 3, 4, 5, 6, 7, 8, 9, 10, 11, 12, 13, 14, 15, 16)
_IS_IDENTITY = _ORDER == tuple(range(len(_ORDER)))

_GRID = 27


def _reorder_block(x_ref, o_ref):
    if _IS_IDENTITY:
        o_ref[...] = x_ref[...]
    else:
        for jj, s in enumerate(_ORDER):
            o_ref[:, jj, :, :] = x_ref[:, s, :, :]


def kernel(x):
    b, n, j, d = x.shape
    xt = jax.lax.transpose(x, (1, 2, 0, 3))
    bn = n // _GRID
    out_t = pl.pallas_call(
        _reorder_block,
        grid=(_GRID,),
        in_specs=[pl.BlockSpec((bn, j, b, d), lambda i: (i, 0, 0, 0))],
        out_specs=pl.BlockSpec((bn, j, b, d), lambda i: (i, 0, 0, 0)),
        out_shape=jax.ShapeDtypeStruct((n, j, b, d), x.dtype),
    )(xt)
    return jax.lax.transpose(out_t, (2, 0, 1, 3))
